# baseline (speedup 1.0000x reference)
import functools

import jax
import jax.numpy as jnp
from jax import lax
from jax.experimental import pallas as pl
from jax.experimental.pallas import tpu as pltpu
from jax.experimental.pallas import tpu_sc as plsc

NUM_NODES = 1000000
NUM_REL = 1000000
DIM = 64
BATCH = 16384

NC = 2
NS = 16
L = 16
NW = NC * NS
B_PER_W = BATCH // NW
IDX_CHUNK = 128
N_CHUNKS = B_PER_W // IDX_CHUNK


def _sc_body(hidx_hbm, ridx_hbm, tidx_hbm, ent_hbm, rel_hbm, out_hbm,
             hidx_v, ridx_v, tidx_v, h_v, r_v, t_v, out_v, sem):
    wid = lax.axis_index("s") * NC + lax.axis_index("c")
    row0 = wid * N_CHUNKS

    pltpu.sync_copy(hidx_hbm.at[pl.ds(row0, N_CHUNKS)], hidx_v)
    pltpu.sync_copy(ridx_hbm.at[pl.ds(row0, N_CHUNKS)], ridx_v)
    pltpu.sync_copy(tidx_hbm.at[pl.ds(row0, N_CHUNKS)], tidx_v)

    copies = []
    for c in range(N_CHUNKS):
        dst = pl.ds(c * IDX_CHUNK, IDX_CHUNK)
        copies.append(pltpu.async_copy(
            ent_hbm.at[hidx_v.at[c]], h_v.at[dst], sem))
        copies.append(pltpu.async_copy(
            rel_hbm.at[ridx_v.at[c]], r_v.at[dst], sem))
        copies.append(pltpu.async_copy(
            ent_hbm.at[tidx_v.at[c]], t_v.at[dst], sem))
    for cp in copies:
        cp.wait()

    lane = lax.iota(jnp.int32, L)

    def group_body(g, carry):
        rows = g * L + lane
        acc = jnp.zeros((L,), jnp.float32)
        for d in range(DIM):
            col = jnp.full((L,), d, jnp.int32)
            gh = plsc.load_gather(h_v, [rows, col])
            gr = plsc.load_gather(r_v, [rows, col])
            gt = plsc.load_gather(t_v, [rows, col])
            acc = acc + jnp.abs(gh + gr - gt)
        out_v[pl.ds(g * L, L)] = -acc
        return carry

    lax.fori_loop(0, B_PER_W // L, group_body, 0)

    pltpu.sync_copy(out_v, out_hbm.at[pl.ds(wid * B_PER_W, B_PER_W)])


@jax.jit
def kernel(triples, entity_emb, relation_emb):
    idx = triples.astype(jnp.int32)
    hidx = idx[:, 0].reshape(BATCH // IDX_CHUNK, IDX_CHUNK)
    ridx = idx[:, 1].reshape(BATCH // IDX_CHUNK, IDX_CHUNK)
    tidx = idx[:, 2].reshape(BATCH // IDX_CHUNK, IDX_CHUNK)

    run = pl.kernel(
        _sc_body,
        mesh=plsc.VectorSubcoreMesh(core_axis_name="c", subcore_axis_name="s"),
        compiler_params=pltpu.CompilerParams(
            needs_layout_passes=False, use_tc_tiling_on_sc=False),
        out_type=jax.ShapeDtypeStruct((BATCH,), jnp.float32),
        scratch_types=[
            pltpu.VMEM((N_CHUNKS, IDX_CHUNK), jnp.int32),
            pltpu.VMEM((N_CHUNKS, IDX_CHUNK), jnp.int32),
            pltpu.VMEM((N_CHUNKS, IDX_CHUNK), jnp.int32),
            pltpu.VMEM((B_PER_W, DIM), jnp.float32),
            pltpu.VMEM((B_PER_W, DIM), jnp.float32),
            pltpu.VMEM((B_PER_W, DIM), jnp.float32),
            pltpu.VMEM((B_PER_W,), jnp.float32),
            pltpu.SemaphoreType.DMA,
        ],
    )
    return run(hidx, ridx, tidx, entity_emb, relation_emb)


# speedup vs baseline: 1.0055x; 1.0055x over previous
import functools

import jax
import jax.numpy as jnp
from jax import lax
from jax.experimental import pallas as pl
from jax.experimental.pallas import tpu as pltpu
from jax.experimental.pallas import tpu_sc as plsc

DIM = 64
BATCH = 16384

NC = 2
NS = 16
L = 16
NW = NC * NS
B_PER_W = BATCH // NW
CHUNK = 128
N_CHUNKS = B_PER_W // CHUNK
PAIR = 2 * DIM


def _sc_body(hp_hbm, rp_hbm, tp_hbm, hq_hbm, rq_hbm, tq_hbm,
             ent2, rel2, out_hbm,
             hp_v, rp_v, tp_v, hq_v, rq_v, tq_v,
             h_a, r_a, t_a, h_b, r_b, t_b, out_v, sem0, sem1):
    wid = lax.axis_index("s") * NC + lax.axis_index("c")
    row0 = wid * N_CHUNKS

    for src, dst in ((hp_hbm, hp_v), (rp_hbm, rp_v), (tp_hbm, tp_v),
                     (hq_hbm, hq_v), (rq_hbm, rq_v), (tq_hbm, tq_v)):
        pltpu.sync_copy(src.at[pl.ds(row0, N_CHUNKS)], dst)

    bufs = [(h_a, r_a, t_a), (h_b, r_b, t_b)]
    sems = [sem0, sem1]

    def fire(c):
        hb, rb, tb = bufs[c % 2]
        sem = sems[c % 2]
        return (pltpu.async_copy(ent2.at[hp_v.at[c]], hb, sem),
                pltpu.async_copy(rel2.at[rp_v.at[c]], rb, sem),
                pltpu.async_copy(ent2.at[tp_v.at[c]], tb, sem))

    def compute(c):
        hb, rb, tb = bufs[c % 2]
        lane = lax.iota(jnp.int32, L)

        def group(g, carry):
            jl = g * L + lane
            sl = pl.ds(g * L, L)
            hq = hq_v[c, sl] * DIM
            rq = rq_v[c, sl] * DIM
            tq = tq_v[c, sl] * DIM
            acc = jnp.zeros((L,), jnp.float32)
            for d in range(DIM):
                gh = plsc.load_gather(hb, [jl, hq + d])
                gr = plsc.load_gather(rb, [jl, rq + d])
                gt = plsc.load_gather(tb, [jl, tq + d])
                acc = acc + jnp.abs(gh + gr - gt)
            out_v[pl.ds(c * CHUNK + g * L, L)] = -acc
            return carry

        lax.fori_loop(0, CHUNK // L, group, 0)

    inflight = fire(0)
    for c in range(N_CHUNKS):
        nxt = fire(c + 1) if c + 1 < N_CHUNKS else ()
        for cp in inflight:
            cp.wait()
        compute(c)
        inflight = nxt

    pltpu.sync_copy(out_v, out_hbm.at[pl.ds(wid * B_PER_W, B_PER_W)])


@jax.jit
def kernel(triples, entity_emb, relation_emb):
    idx = triples.astype(jnp.int32)
    grids = []
    for k in range(3):
        col = idx[:, k]
        grids.append(jnp.right_shift(col, 1).reshape(BATCH // CHUNK, CHUNK))
        grids.append(jnp.bitwise_and(col, 1).reshape(BATCH // CHUNK, CHUNK))
    hp, hq, rp, rq, tp, tq = grids
    ent2 = entity_emb.reshape(entity_emb.shape[0] // 2, PAIR)
    rel2 = relation_emb.reshape(relation_emb.shape[0] // 2, PAIR)

    run = pl.kernel(
        _sc_body,
        mesh=plsc.VectorSubcoreMesh(core_axis_name="c", subcore_axis_name="s"),
        out_type=jax.ShapeDtypeStruct((BATCH,), jnp.float32),
        scratch_types=[
            pltpu.VMEM((N_CHUNKS, CHUNK), jnp.int32),
            pltpu.VMEM((N_CHUNKS, CHUNK), jnp.int32),
            pltpu.VMEM((N_CHUNKS, CHUNK), jnp.int32),
            pltpu.VMEM((N_CHUNKS, CHUNK), jnp.int32),
            pltpu.VMEM((N_CHUNKS, CHUNK), jnp.int32),
            pltpu.VMEM((N_CHUNKS, CHUNK), jnp.int32),
            pltpu.VMEM((CHUNK, PAIR), jnp.float32),
            pltpu.VMEM((CHUNK, PAIR), jnp.float32),
            pltpu.VMEM((CHUNK, PAIR), jnp.float32),
            pltpu.VMEM((CHUNK, PAIR), jnp.float32),
            pltpu.VMEM((CHUNK, PAIR), jnp.float32),
            pltpu.VMEM((CHUNK, PAIR), jnp.float32),
            pltpu.VMEM((B_PER_W,), jnp.float32),
            pltpu.SemaphoreType.DMA,
            pltpu.SemaphoreType.DMA,
        ],
        compiler_params=pltpu.CompilerParams(needs_layout_passes=False),
    )
    return run(hp, rp, tp, hq, rq, tq, ent2, rel2)
